# baseline (speedup 1.0000x reference)
import functools

import jax
import jax.numpy as jnp
from jax import lax
from jax.experimental import pallas as pl
from jax.experimental.pallas import tpu as pltpu
from jax.experimental.pallas import tpu_sc as plsc

B = 4096
NF = 26
V = 100000
D = 16
ND = 13
H1, H2 = 256, 128
NV = NF * V
BNF = B * NF
NC, NS = 2, 16
NW = NC * NS
PER_W = BNF // NW
NDMA = 16


TL = 65536
NVP = NV + 64
DSTEP = -(-NVP // TL)
TAILW = NVP - (DSTEP - 1) * TL


DU = 1024
NU = NVP // DU
NQ = -(-(NU + 1) // NW)


def _detile(emb_t):
    mesh = plsc.VectorSubcoreMesh(core_axis_name="c", subcore_axis_name="s")

    @functools.partial(
        pl.kernel,
        mesh=mesh,
        compiler_params=pltpu.CompilerParams(use_tc_tiling_on_sc=True,
                                             disable_bounds_checks=True),
        out_type=jax.ShapeDtypeStruct((D * NVP,), jnp.float32),
        scratch_types=[
            pltpu.VMEM((D, DU), jnp.float32),
            pltpu.VMEM((D, DU), jnp.float32),
            pltpu.SemaphoreType.DMA,
            pltpu.SemaphoreType.DMA,
            pltpu.SemaphoreType.DMA,
            pltpu.SemaphoreType.DMA,
        ],
    )
    def detile_k(in_hbm, out_hbm, bufa, bufb, sia, sib, soa, sob):
        wid = lax.axis_index("s") * NC + lax.axis_index("c")

        def load(q, buf, si):
            u = wid + NW * q

            @pl.when(u < NU)
            def _():
                pltpu.async_copy(in_hbm.at[:, pl.ds(u * DU, DU)], buf, si)

            @pl.when(u == NU)
            def _():
                pltpu.async_copy(in_hbm.at[:, pl.ds(u * DU, 128)],
                                 buf.at[:, pl.ds(0, 128)], si)

        def store(q, buf, si, so):
            u = wid + NW * q

            @pl.when(u < NU)
            def _():
                pltpu.make_async_copy(in_hbm.at[:, pl.ds(0, DU)], buf,
                                      si).wait()
                for c in range(D):
                    pltpu.async_copy(buf.at[c],
                                     out_hbm.at[pl.ds(c * NVP + u * DU, DU)],
                                     so)

            @pl.when(u == NU)
            def _():
                pltpu.make_async_copy(in_hbm.at[:, pl.ds(0, 128)],
                                      buf.at[:, pl.ds(0, 128)], si).wait()
                for c in range(D):
                    pltpu.async_copy(buf.at[c, pl.ds(0, 128)],
                                     out_hbm.at[pl.ds(c * NVP + u * DU, 128)],
                                     so)

        def drain(q, buf, so):
            u = wid + NW * q

            @pl.when(u < NU)
            def _():
                for c in range(D):
                    pltpu.make_async_copy(buf.at[c],
                                          out_hbm.at[pl.ds(0, DU)],
                                          so).wait()

            @pl.when(u == NU)
            def _():
                for c in range(D):
                    pltpu.make_async_copy(buf.at[c, pl.ds(0, 128)],
                                          out_hbm.at[pl.ds(0, 128)],
                                          so).wait()

        load(0, bufa, sia)
        load(1, bufb, sib)

        @pl.loop(0, NQ // 2)
        def _(g):
            q = g * 2
            store(q, bufa, sia, soa)
            store(q + 1, bufb, sib, sob)

            @pl.when(g < NQ // 2 - 1)
            def _():
                drain(q, bufa, soa)
                load(q + 2, bufa, sia)
                drain(q + 1, bufb, sob)
                load(q + 3, bufb, sib)

        drain(NQ - 2, bufa, soa)
        drain(NQ - 1, bufb, sob)

    return detile_k(emb_t)


CHUNK = 128
NCHUNK = PER_W // CHUNK
NVG = NV // D
NVGP = NVG + 4
NT = D + 1
RING = 8


def _sc_gather(emb16, lin16, idxh3d, idxlo3d):
    mesh = plsc.VectorSubcoreMesh(core_axis_name="c", subcore_axis_name="s")

    @functools.partial(
        pl.kernel,
        mesh=mesh,
        compiler_params=pltpu.CompilerParams(use_tc_tiling_on_sc=False,
                                             needs_layout_passes=False),
        out_type=[
            jax.ShapeDtypeStruct((D, BNF), jnp.float32),
            jax.ShapeDtypeStruct((BNF,), jnp.float32),
        ],
        scratch_types=[
            pltpu.VMEM((NCHUNK, CHUNK), jnp.int32),
            pltpu.VMEM((NCHUNK, CHUNK), jnp.int32),
            pltpu.VMEM((RING, CHUNK, D), jnp.float32),
            pltpu.VMEM((NT, PER_W), jnp.float32),
            pltpu.SemaphoreType.DMA((RING,)),
        ],
    )
    def gather_k(emb_hbm, lin_hbm, idxh_hbm, idxlo_hbm, e_out, l_out,
                 idxh_v, idxlo_v, bufs, vals, sems):
        wid = lax.axis_index("s") * NC + lax.axis_index("c")
        base = wid * PER_W
        pltpu.sync_copy(idxh_hbm.at[wid], idxh_v)
        pltpu.sync_copy(idxlo_hbm.at[wid], idxlo_v)
        iota16 = lax.iota(jnp.int32, 16)

        def do_table(t, tbl):
            def fire(k, j):
                pltpu.async_copy(tbl.at[idxh_v.at[k]], bufs.at[j], sems.at[j])

            def extract(k, j):
                pltpu.make_async_copy(lin_hbm.at[pl.ds(0, CHUNK)],
                                      bufs.at[j], sems.at[j]).wait()

                @pl.loop(0, CHUNK // 16)
                def _(i):
                    lo16 = idxlo_v[k, pl.ds(i * 16, 16)]
                    rows = iota16 + i * 16
                    val = plsc.load_gather(bufs.at[j], [rows, lo16])
                    vals[t, pl.ds(k * CHUNK + i * 16, 16)] = val

            for j in range(RING):
                fire(j, j)

            @pl.loop(0, NCHUNK // RING - 1)
            def _(g):
                k0 = g * RING
                for j in range(RING):
                    extract(k0 + j, j)
                    fire(k0 + RING + j, j)

            tail = (NCHUNK // RING - 1) * RING
            for j in range(RING):
                extract(tail + j, j)
                if tail + RING + j < NCHUNK:
                    fire(tail + RING + j, j)
            for j in range(NCHUNK - tail - RING):
                extract(tail + RING + j, j)

        @pl.loop(0, D)
        def _(t):
            do_table(t, emb_hbm.at[pl.ds(t * NVGP, NVG)])

        do_table(D, lin_hbm)

        pltpu.sync_copy(vals.at[pl.ds(0, D)], e_out.at[:, pl.ds(base, PER_W)])
        pltpu.sync_copy(vals.at[D], l_out.at[pl.ds(base, PER_W)])

    return gather_k(emb16, lin16, idxh3d, idxlo3d)


BC = 512
NCH = B // BC

_HI = jax.lax.Precision.HIGHEST


def _dot(a, b):
    return jnp.dot(a, b, precision=_HI, preferred_element_type=jnp.float32)


def _tc_body(dense_ref, e_ref, lin_ref,
             Wd_ref, W1a_ref, W1bs_ref,
             b1_ref, g1_ref, be1_ref, W2_ref, b2_ref, g2_ref, be2_ref,
             Wo_ref, bias_ref, out_ref,
             z1_scr, z2_scr, a1s, a1q, a2s, a2q):
    p = pl.program_id(0)
    i = pl.program_id(1)

    @pl.when((p == 0) & (i == 0))
    def _():
        a1s[...] = jnp.zeros_like(a1s)
        a1q[...] = jnp.zeros_like(a1q)
        a2s[...] = jnp.zeros_like(a2s)
        a2q[...] = jnp.zeros_like(a2q)

    @pl.when(p == 0)
    def _():
        z1 = _dot(dense_ref[...], W1a_ref[...]) + b1_ref[...]
        for c in range(D):
            z1 = z1 + _dot(e_ref[c], W1bs_ref[c])
        z1_scr[pl.ds(i * BC, BC), :] = z1
        a1s[...] += jnp.sum(z1, axis=0, keepdims=True)
        a1q[...] += jnp.sum(z1 * z1, axis=0, keepdims=True)
        out_ref[...] = jnp.zeros_like(out_ref)

    @pl.when(p == 1)
    def _():
        m1 = a1s[...] * (1.0 / B)
        v1 = a1q[...] * (1.0 / B) - m1 * m1
        z1 = z1_scr[pl.ds(i * BC, BC), :]
        h1 = jnp.maximum((z1 - m1) * lax.rsqrt(v1 + 1e-5) * g1_ref[...]
                         + be1_ref[...], 0.0)
        z2 = _dot(h1, W2_ref[...]) + b2_ref[...]
        z2_scr[pl.ds(i * BC, BC), :] = z2
        a2s[...] += jnp.sum(z2, axis=0, keepdims=True)
        a2q[...] += jnp.sum(z2 * z2, axis=0, keepdims=True)
        out_ref[...] = jnp.zeros_like(out_ref)

    @pl.when(p == 2)
    def _():
        linear = (_dot(dense_ref[...], Wd_ref[...])
                  + jnp.sum(lin_ref[...], axis=1, keepdims=True))
        fm = jnp.zeros((BC, 1), jnp.float32)
        for c in range(D):
            ec = e_ref[c]
            sc = jnp.sum(ec, axis=1, keepdims=True)
            ssc = jnp.sum(ec * ec, axis=1, keepdims=True)
            fm = fm + (sc * sc - ssc)
        fm = 0.5 * fm

        m2 = a2s[...] * (1.0 / B)
        v2 = a2q[...] * (1.0 / B) - m2 * m2
        z2 = z2_scr[pl.ds(i * BC, BC), :]
        h2 = jnp.maximum((z2 - m2) * lax.rsqrt(v2 + 1e-5) * g2_ref[...]
                         + be2_ref[...], 0.0)
        deep = _dot(h2, Wo_ref[...])
        out_ref[...] = linear + fm + deep + bias_ref[...]


def _tc_forward(dense_features, e3, lin_vals, W_dense, b_dense,
                W1, b1, g1, be1, W2, b2, g2, be2, W_out, b_out,
                interpret=False):
    W1bs = W1[ND:].reshape(NF, D, H1).transpose(1, 0, 2)
    bias = (b_dense + b_out).reshape(1, 1).astype(jnp.float32)

    chunk = lambda p, i: (i, 0)
    chunk3 = lambda p, i: (0, i, 0)
    whole = lambda p, i: (0, 0)
    whole3 = lambda p, i: (0, 0, 0)
    out = pl.pallas_call(
        _tc_body,
        grid=(3, NCH),
        in_specs=[
            pl.BlockSpec((BC, ND), chunk),
            pl.BlockSpec((D, BC, NF), chunk3),
            pl.BlockSpec((BC, NF), chunk),
            pl.BlockSpec((ND, 1), whole),
            pl.BlockSpec((ND, H1), whole),
            pl.BlockSpec((D, NF, H1), whole3),
            pl.BlockSpec((1, H1), whole),
            pl.BlockSpec((1, H1), whole),
            pl.BlockSpec((1, H1), whole),
            pl.BlockSpec((H1, H2), whole),
            pl.BlockSpec((1, H2), whole),
            pl.BlockSpec((1, H2), whole),
            pl.BlockSpec((1, H2), whole),
            pl.BlockSpec((H2, 1), whole),
            pl.BlockSpec((1, 1), whole),
        ],
        out_specs=pl.BlockSpec((BC, 1), chunk),
        scratch_shapes=[
            pltpu.VMEM((B, H1), jnp.float32),
            pltpu.VMEM((B, H2), jnp.float32),
            pltpu.VMEM((1, H1), jnp.float32),
            pltpu.VMEM((1, H1), jnp.float32),
            pltpu.VMEM((1, H2), jnp.float32),
            pltpu.VMEM((1, H2), jnp.float32),
        ],
        out_shape=jax.ShapeDtypeStruct((B, 1), jnp.float32),
        interpret=interpret,
    )(dense_features, e3, lin_vals,
      W_dense.reshape(ND, 1), W1[:ND], W1bs, b1.reshape(1, H1),
      g1.reshape(1, H1), be1.reshape(1, H1), W2, b2.reshape(1, H2),
      g2.reshape(1, H2), be2.reshape(1, H2), W_out.reshape(H2, 1), bias)
    return out.reshape(B)


def kernel(dense_features, sparse_features, emb_table, lin_table, W_dense,
           b_dense, W1, b1, g1, be1, W2, b2, g2, be2, W_out, b_out):
    offsets = sparse_features + jnp.arange(NF, dtype=jnp.int32)[None, :] * V
    idxh3d = (offsets // D).reshape(NW, NCHUNK, CHUNK)
    idxlo3d = (offsets % D).reshape(NW, NCHUNK, CHUNK)

    emb16 = _detile(emb_table.T).reshape(D * NVGP, D)
    lin16 = lin_table.reshape(NVG, D)
    e_cm, l_rows = _sc_gather(emb16, lin16, idxh3d, idxlo3d)
    e3 = e_cm.reshape(D, B, NF)
    lin_vals = l_rows.reshape(B, NF)

    return _tc_forward(dense_features, e3, lin_vals, W_dense, b_dense,
                       W1, b1, g1, be1, W2, b2, g2, be2, W_out, b_out)


# speedup vs baseline: 1.0200x; 1.0200x over previous
import functools

import jax
import jax.numpy as jnp
from jax import lax
from jax.experimental import pallas as pl
from jax.experimental.pallas import tpu as pltpu
from jax.experimental.pallas import tpu_sc as plsc

B = 4096
NF = 26
V = 100000
D = 16
ND = 13
H1, H2 = 256, 128
NV = NF * V
BNF = B * NF
NC, NS = 2, 16
NW = NC * NS
PER_W = BNF // NW

TL = 65536
NVP = NV + 64
DSTEP = -(-NVP // TL)
TAILW = NVP - (DSTEP - 1) * TL


def _detile_body(in_ref, out_hbm, sem):
    i = pl.program_id(0)

    @pl.when(i < DSTEP - 1)
    def _():
        for c in range(D):
            pltpu.async_copy(in_ref.at[c],
                             out_hbm.at[pl.ds(c * NVP + i * TL, TL)], sem)
        for c in range(D):
            pltpu.make_async_copy(in_ref.at[c],
                                  out_hbm.at[pl.ds(0, TL)], sem).wait()

    @pl.when(i == DSTEP - 1)
    def _():
        for c in range(D):
            pltpu.async_copy(in_ref.at[c, pl.ds(0, TAILW)],
                             out_hbm.at[pl.ds(c * NVP + i * TL, TAILW)], sem)
        for c in range(D):
            pltpu.make_async_copy(in_ref.at[c, pl.ds(0, TAILW)],
                                  out_hbm.at[pl.ds(0, TAILW)], sem).wait()


def _detile(emb_t):
    return pl.pallas_call(
        _detile_body,
        grid=(DSTEP,),
        in_specs=[pl.BlockSpec((D, TL), lambda i: (0, i))],
        out_specs=pl.BlockSpec(memory_space=pl.ANY),
        out_shape=jax.ShapeDtypeStruct((D * NVP,), jnp.float32),
        scratch_shapes=[pltpu.SemaphoreType.DMA],
    )(emb_t)


CHUNK = 128
NCHUNK = PER_W // CHUNK
NVG = NV // D
NVGP = NVG + 4
NT = D + 1
RING = 8


def _sc_gather(emb16, lin16, idxh3d, idxlo3d):
    mesh = plsc.VectorSubcoreMesh(core_axis_name="c", subcore_axis_name="s")

    @functools.partial(
        pl.kernel,
        mesh=mesh,
        compiler_params=pltpu.CompilerParams(use_tc_tiling_on_sc=False,
                                             needs_layout_passes=False),
        out_type=[
            jax.ShapeDtypeStruct((D, BNF), jnp.float32),
            jax.ShapeDtypeStruct((BNF,), jnp.float32),
        ],
        scratch_types=[
            pltpu.VMEM((NCHUNK, CHUNK), jnp.int32),
            pltpu.VMEM((NCHUNK, CHUNK), jnp.int32),
            pltpu.VMEM((RING, CHUNK, D), jnp.float32),
            pltpu.VMEM((NT, PER_W), jnp.float32),
            pltpu.SemaphoreType.DMA((RING,)),
        ],
    )
    def gather_k(emb_hbm, lin_hbm, idxh_hbm, idxlo_hbm, e_out, l_out,
                 idxh_v, idxlo_v, bufs, vals, sems):
        wid = lax.axis_index("s") * NC + lax.axis_index("c")
        base = wid * PER_W
        pltpu.sync_copy(idxh_hbm.at[wid], idxh_v)
        pltpu.sync_copy(idxlo_hbm.at[wid], idxlo_v)
        iota16 = lax.iota(jnp.int32, 16)

        def do_table(t, tbl):
            def fire(k, j):
                pltpu.async_copy(tbl.at[idxh_v.at[k]], bufs.at[j], sems.at[j])

            def extract(k, j):
                pltpu.make_async_copy(lin_hbm.at[pl.ds(0, CHUNK)],
                                      bufs.at[j], sems.at[j]).wait()

                @pl.loop(0, CHUNK // 16)
                def _(i):
                    lo16 = idxlo_v[k, pl.ds(i * 16, 16)]
                    rows = iota16 + i * 16
                    val = plsc.load_gather(bufs.at[j], [rows, lo16])
                    vals[t, pl.ds(k * CHUNK + i * 16, 16)] = val

            for j in range(RING):
                fire(j, j)

            @pl.loop(0, NCHUNK // RING - 1)
            def _(g):
                k0 = g * RING
                for j in range(RING):
                    extract(k0 + j, j)
                    fire(k0 + RING + j, j)

            tail = (NCHUNK // RING - 1) * RING
            for j in range(RING):
                extract(tail + j, j)
                if tail + RING + j < NCHUNK:
                    fire(tail + RING + j, j)
            for j in range(NCHUNK - tail - RING):
                extract(tail + RING + j, j)

        @pl.loop(0, D)
        def _(t):
            do_table(t, emb_hbm.at[pl.ds(t * NVGP, NVG)])

        do_table(D, lin_hbm)

        pltpu.sync_copy(vals.at[pl.ds(0, D)], e_out.at[:, pl.ds(base, PER_W)])
        pltpu.sync_copy(vals.at[D], l_out.at[pl.ds(base, PER_W)])

    return gather_k(emb16, lin16, idxh3d, idxlo3d)


BC = 512
NCH = B // BC

_HI = jax.lax.Precision.HIGHEST


def _dot(a, b):
    return jnp.dot(a, b, precision=_HI, preferred_element_type=jnp.float32)


def _tc_body(dense_ref, e_ref, lin_ref,
             Wd_ref, W1a_ref, W1bs_ref,
             b1_ref, g1_ref, be1_ref, W2_ref, b2_ref, g2_ref, be2_ref,
             Wo_ref, bias_ref, out_ref,
             z1_scr, z2_scr, a1s, a1q, a2s, a2q):
    p = pl.program_id(0)
    i = pl.program_id(1)

    @pl.when((p == 0) & (i == 0))
    def _():
        a1s[...] = jnp.zeros_like(a1s)
        a1q[...] = jnp.zeros_like(a1q)
        a2s[...] = jnp.zeros_like(a2s)
        a2q[...] = jnp.zeros_like(a2q)

    @pl.when(p == 0)
    def _():
        z1 = _dot(dense_ref[...], W1a_ref[...]) + b1_ref[...]
        for c in range(D):
            z1 = z1 + _dot(e_ref[c], W1bs_ref[c])
        z1_scr[pl.ds(i * BC, BC), :] = z1
        a1s[...] += jnp.sum(z1, axis=0, keepdims=True)
        a1q[...] += jnp.sum(z1 * z1, axis=0, keepdims=True)
        out_ref[...] = jnp.zeros_like(out_ref)

    @pl.when(p == 1)
    def _():
        m1 = a1s[...] * (1.0 / B)
        v1 = a1q[...] * (1.0 / B) - m1 * m1
        z1 = z1_scr[pl.ds(i * BC, BC), :]
        h1 = jnp.maximum((z1 - m1) * lax.rsqrt(v1 + 1e-5) * g1_ref[...]
                         + be1_ref[...], 0.0)
        z2 = _dot(h1, W2_ref[...]) + b2_ref[...]
        z2_scr[pl.ds(i * BC, BC), :] = z2
        a2s[...] += jnp.sum(z2, axis=0, keepdims=True)
        a2q[...] += jnp.sum(z2 * z2, axis=0, keepdims=True)
        out_ref[...] = jnp.zeros_like(out_ref)

    @pl.when(p == 2)
    def _():
        linear = (_dot(dense_ref[...], Wd_ref[...])
                  + jnp.sum(lin_ref[...], axis=1, keepdims=True))
        fm = jnp.zeros((BC, 1), jnp.float32)
        for c in range(D):
            ec = e_ref[c]
            sc = jnp.sum(ec, axis=1, keepdims=True)
            ssc = jnp.sum(ec * ec, axis=1, keepdims=True)
            fm = fm + (sc * sc - ssc)
        fm = 0.5 * fm

        m2 = a2s[...] * (1.0 / B)
        v2 = a2q[...] * (1.0 / B) - m2 * m2
        z2 = z2_scr[pl.ds(i * BC, BC), :]
        h2 = jnp.maximum((z2 - m2) * lax.rsqrt(v2 + 1e-5) * g2_ref[...]
                         + be2_ref[...], 0.0)
        deep = _dot(h2, Wo_ref[...])
        out_ref[...] = linear + fm + deep + bias_ref[...]


def _tc_forward(dense_features, e3, lin_vals, W_dense, b_dense,
                W1, b1, g1, be1, W2, b2, g2, be2, W_out, b_out):
    W1bs = W1[ND:].reshape(NF, D, H1).transpose(1, 0, 2)
    bias = (b_dense + b_out).reshape(1, 1).astype(jnp.float32)

    chunk = lambda p, i: (i, 0)
    chunk3 = lambda p, i: (0, i, 0)
    whole = lambda p, i: (0, 0)
    whole3 = lambda p, i: (0, 0, 0)
    out = pl.pallas_call(
        _tc_body,
        grid=(3, NCH),
        in_specs=[
            pl.BlockSpec((BC, ND), chunk),
            pl.BlockSpec((D, BC, NF), chunk3),
            pl.BlockSpec((BC, NF), chunk),
            pl.BlockSpec((ND, 1), whole),
            pl.BlockSpec((ND, H1), whole),
            pl.BlockSpec((D, NF, H1), whole3),
            pl.BlockSpec((1, H1), whole),
            pl.BlockSpec((1, H1), whole),
            pl.BlockSpec((1, H1), whole),
            pl.BlockSpec((H1, H2), whole),
            pl.BlockSpec((1, H2), whole),
            pl.BlockSpec((1, H2), whole),
            pl.BlockSpec((1, H2), whole),
            pl.BlockSpec((H2, 1), whole),
            pl.BlockSpec((1, 1), whole),
        ],
        out_specs=pl.BlockSpec((BC, 1), chunk),
        scratch_shapes=[
            pltpu.VMEM((B, H1), jnp.float32),
            pltpu.VMEM((B, H2), jnp.float32),
            pltpu.VMEM((1, H1), jnp.float32),
            pltpu.VMEM((1, H1), jnp.float32),
            pltpu.VMEM((1, H2), jnp.float32),
            pltpu.VMEM((1, H2), jnp.float32),
        ],
        out_shape=jax.ShapeDtypeStruct((B, 1), jnp.float32),
    )(dense_features, e3, lin_vals,
      W_dense.reshape(ND, 1), W1[:ND], W1bs, b1.reshape(1, H1),
      g1.reshape(1, H1), be1.reshape(1, H1), W2, b2.reshape(1, H2),
      g2.reshape(1, H2), be2.reshape(1, H2), W_out.reshape(H2, 1), bias)
    return out.reshape(B)


def kernel(dense_features, sparse_features, emb_table, lin_table, W_dense,
           b_dense, W1, b1, g1, be1, W2, b2, g2, be2, W_out, b_out):
    offsets = sparse_features + jnp.arange(NF, dtype=jnp.int32)[None, :] * V
    idxh3d = (offsets // D).reshape(NW, NCHUNK, CHUNK)
    idxlo3d = (offsets % D).reshape(NW, NCHUNK, CHUNK)

    emb16 = _detile(emb_table.T).reshape(D * NVGP, D)
    lin16 = lin_table.reshape(NVG, D)
    e_cm, l_rows = _sc_gather(emb16, lin16, idxh3d, idxlo3d)
    e3 = e_cm.reshape(D, B, NF)
    lin_vals = l_rows.reshape(B, NF)

    return _tc_forward(dense_features, e3, lin_vals, W_dense, b_dense,
                       W1, b1, g1, be1, W2, b2, g2, be2, W_out, b_out)
